# Initial kernel scaffold; baseline (speedup 1.0000x reference)
import jax
import jax.numpy as jnp
from jax.experimental import pallas as pl


def kernel(hidden_states, gate_w):
    raise NotImplementedError("write your pallas kernel here")


# speedup vs baseline: 1.1731x; 1.1731x over previous
import jax
import jax.numpy as jnp
from jax.experimental import pallas as pl

_TOP_K = 2
_TB = 1024


def _routing_kernel(x_ref, gw_ref, z_ref, m_ref):
    z_ref[...] = jnp.zeros_like(z_ref)
    lt = jax.lax.dot_general(
        gw_ref[...], x_ref[...],
        dimension_numbers=(((1,), (1,)), ((), ())),
        preferred_element_type=jnp.float32,
    )
    E = lt.shape[0]
    eidx = jax.lax.broadcasted_iota(jnp.int32, lt.shape, 0)
    m1 = jnp.max(lt, axis=0, keepdims=True)
    i1 = jnp.min(jnp.where(lt == m1, eidx, E), axis=0, keepdims=True)
    lt2 = jnp.where(eidx == i1, -jnp.inf, lt)
    m2 = jnp.max(lt2, axis=0, keepdims=True)
    i2 = jnp.min(jnp.where(lt2 == m2, eidx, E), axis=0, keepdims=True)
    e3 = jax.lax.broadcasted_iota(jnp.int32, m_ref.shape, 0)
    k3 = jax.lax.broadcasted_iota(jnp.int32, m_ref.shape, 1)
    sel = jnp.where(k3 == 0, i1[None], i2[None])
    m_ref[...] = (e3 == sel).astype(jnp.float32)


def kernel(hidden_states, gate_w):
    b, s, h = hidden_states.shape
    t = b * s
    e = gate_w.shape[0]
    x = hidden_states.reshape(t, h)
    grid = (t // _TB,)
    z, m = pl.pallas_call(
        _routing_kernel,
        grid=grid,
        in_specs=[
            pl.BlockSpec((_TB, h), lambda i: (i, 0)),
            pl.BlockSpec((e, h), lambda i: (0, 0)),
        ],
        out_specs=[
            pl.BlockSpec((_TB, h), lambda i: (i, 0)),
            pl.BlockSpec((e, _TOP_K, _TB), lambda i: (0, 0, i)),
        ],
        out_shape=[
            jax.ShapeDtypeStruct((t, h), jnp.float32),
            jax.ShapeDtypeStruct((e, _TOP_K, t), jnp.float32),
        ],
    )(x, gate_w)
    return (z, m)
